# Initial kernel scaffold; baseline (speedup 1.0000x reference)
import jax
import jax.numpy as jnp
from jax.experimental import pallas as pl


def kernel(x, edge_index, W_enc, b_enc, W_g1, b_g1, ln1_g, ln1_b, W_g2, b_g2, ln2_g, ln2_b, W_ref, b_ref, W_q, b_q, W_f, b_f, W_v, b_v):
    raise NotImplementedError("write your pallas kernel here")


# speedup vs baseline: 8.8705x; 8.8705x over previous
import functools

import jax
import jax.numpy as jnp
from jax import lax
from jax.experimental import pallas as pl
from jax.experimental.pallas import tpu as pltpu
from jax.experimental.pallas import tpu_sc as plsc

N = 100000
E = 1600000
D = 32
EPS = 1e-5

NC = 2
NS = 16
HALF = N // NC
TBL = 50048
RPT = TBL // NS
WB_LAST = HALF - (NS - 1) * RPT
ZR = 136
NZ = RPT // ZR

CHUNK = 80
CPB = 8
BLK = CPB * CHUNK
EROWS = 20480
PAD_E = EROWS * CHUNK - E
ROWS_PT = EROWS // NS
NBLK = ROWS_PT // CPB

_MESH = plsc.VectorSubcoreMesh(
    core_axis_name="c", subcore_axis_name="s", num_cores=NC, num_subcores=NS)
_SC_PARAMS = pltpu.CompilerParams(use_tc_tiling_on_sc=False)


def _dst_local(draw, dloc, k, nbase):
  r = draw.at[k]
  w = dloc.at[k]
  for j in range(CHUNK // 16):
    v = r[pl.ds(j * 16, 16)]
    lo = v - nbase
    ok = (lo >= 0) & (lo < HALF)
    w[pl.ds(j * 16, 16)] = jnp.where(ok, lo, HALF)


def _zero_table(agg, zv, z_h, s):
  pltpu.sync_copy(z_h, zv)
  for t in range(NZ):
    pltpu.sync_copy(zv, agg.at[pl.ds(s * RPT + t * ZR, ZR)])


def _writeback(agg, out_h, s, nbase):
  @pl.when(s < NS - 1)
  def _():
    pltpu.sync_copy(agg.at[pl.ds(s * RPT, RPT)],
                    out_h.at[pl.ds(nbase + s * RPT, RPT)])

  @pl.when(s == NS - 1)
  def _():
    pltpu.sync_copy(agg.at[pl.ds((NS - 1) * RPT, WB_LAST)],
                    out_h.at[pl.ds(nbase + (NS - 1) * RPT, WB_LAST)])


@functools.partial(
    pl.kernel,
    out_type=jax.ShapeDtypeStruct((N, 8), jnp.float32),
    mesh=_MESH,
    compiler_params=_SC_PARAMS,
    scratch_types=[
        pltpu.VMEM_SHARED((TBL, 8), jnp.float32),
        pltpu.VMEM((ZR, 8), jnp.float32),
        pltpu.VMEM((CHUNK, 8), jnp.float32),
        pltpu.VMEM((CPB, CHUNK), jnp.int32),
        pltpu.VMEM((CPB, CHUNK), jnp.int32),
    ])
def _sc_degree(dst2_h, z_h, ones_h, deg_h, agg, zv, ov, draw, dloc):
  c = lax.axis_index("c")
  s = lax.axis_index("s")
  nbase = c * HALF
  pltpu.sync_copy(ones_h, ov)
  _zero_table(agg, zv, z_h, s)
  plsc.subcore_barrier()

  def blk(b, carry):
    roff = s * ROWS_PT + b * CPB
    pltpu.sync_copy(dst2_h.at[pl.ds(roff, CPB)], draw)
    for k in range(CPB):
      _dst_local(draw, dloc, k, nbase)
    for k in range(CPB):
      pltpu.sync_copy(ov, agg.at[dloc.at[k]], add=True)
    return carry

  lax.fori_loop(0, NBLK, blk, 0)
  plsc.subcore_barrier()
  _writeback(agg, deg_h, s, nbase)


@functools.partial(
    pl.kernel,
    out_type=jax.ShapeDtypeStruct((N, D), jnp.float32),
    mesh=_MESH,
    compiler_params=_SC_PARAMS,
    scratch_types=[
        pltpu.VMEM_SHARED((TBL, D), jnp.float32),
        pltpu.VMEM((ZR, D), jnp.float32),
        pltpu.VMEM((CPB, CHUNK), jnp.int32),
        pltpu.VMEM((CPB, CHUNK), jnp.int32),
        pltpu.VMEM((CPB, CHUNK), jnp.int32),
        pltpu.VMEM((CHUNK, D), jnp.float32),
        pltpu.SemaphoreType.DMA,
    ])
def _sc_agg(src2_h, dst2_h, t_h, z_h, S_h, agg, zv, sidx, draw, dloc, rows,
            sem):
  c = lax.axis_index("c")
  s = lax.axis_index("s")
  nbase = c * HALF
  _zero_table(agg, zv, z_h, s)
  plsc.subcore_barrier()

  def blk(b, carry):
    roff = s * ROWS_PT + b * CPB
    pltpu.sync_copy(src2_h.at[pl.ds(roff, CPB)], sidx)
    pltpu.sync_copy(dst2_h.at[pl.ds(roff, CPB)], draw)
    for k in range(CPB):
      _dst_local(draw, dloc, k, nbase)
    for k in range(CPB):
      pltpu.async_copy(t_h.at[sidx.at[k]], rows, sem).wait()
      pltpu.sync_copy(rows, agg.at[dloc.at[k]], add=True)
    return carry

  lax.fori_loop(0, NBLK, blk, 0)
  plsc.subcore_barrier()
  _writeback(agg, S_h, s, nbase)


R = 10000


def _dis(deg_blk):
  return lax.rsqrt(deg_blk[:, 0:1] + 1.0)


def _enc_body(x_r, deg_r, we_r, be_r, wg_r, t1_r):
  h = jnp.dot(x_r[...], we_r[...], preferred_element_type=jnp.float32)
  h = h + be_r[...]
  t1_r[...] = jnp.dot(h, wg_r[...],
                      preferred_element_type=jnp.float32) * _dis(deg_r[...])


def _layer_norm(u, g, b):
  mu = jnp.mean(u, axis=-1, keepdims=True)
  var = jnp.mean((u - mu) ** 2, axis=-1, keepdims=True)
  return (u - mu) * lax.rsqrt(var + EPS) * g + b


def _mid_body(S_r, t_r, deg_r, bg_r, g_r, b_r, wg2_r, t2_r):
  dis = _dis(deg_r[...])
  u = (S_r[...] + t_r[...]) * dis + bg_r[...]
  h = jnp.maximum(_layer_norm(u, g_r[...], b_r[...]), 0.0)
  t2_r[...] = jnp.dot(h, wg2_r[...], preferred_element_type=jnp.float32) * dis


def _out_body(S_r, t_r, deg_r, bg_r, g_r, b_r, wref_r, bref_r, wcat_r, bcat_r,
              o_r):
  dis = _dis(deg_r[...])
  u = (S_r[...] + t_r[...]) * dis + bg_r[...]
  h = jnp.maximum(_layer_norm(u, g_r[...], b_r[...]), 0.0)
  belief = jnp.maximum(
      jnp.dot(h, wref_r[...], preferred_element_type=jnp.float32) + bref_r[...],
      0.0)
  o_r[...] = jnp.dot(belief, wcat_r[...],
                     preferred_element_type=jnp.float32) + bcat_r[...]


def _row_spec(cols):
  return pl.BlockSpec((R, cols), lambda i: (i, 0))


def _full_spec(r, c):
  return pl.BlockSpec((r, c), lambda i: (0, 0))


def _tc_call(body, in_specs, out_cols, args):
  return pl.pallas_call(
      body,
      grid=(N // R,),
      in_specs=in_specs,
      out_specs=_row_spec(out_cols),
      out_shape=jax.ShapeDtypeStruct((N, out_cols), jnp.float32),
  )(*args)


def kernel(x, edge_index, W_enc, b_enc, W_g1, b_g1, ln1_g, ln1_b, W_g2, b_g2,
           ln2_g, ln2_b, W_ref, b_ref, W_q, b_q, W_f, b_f, W_v, b_v):
  src = edge_index[0].astype(jnp.int32)
  dst = edge_index[1].astype(jnp.int32)
  src2 = jnp.concatenate([src, jnp.zeros((PAD_E,), jnp.int32)]
                         ).reshape(EROWS, CHUNK)
  dst2 = jnp.concatenate([dst, jnp.full((PAD_E,), N, jnp.int32)]
                         ).reshape(EROWS, CHUNK)
  z32 = jnp.zeros((ZR, D), jnp.float32)
  z8 = jnp.zeros((ZR, 8), jnp.float32)
  ones8 = jnp.ones((CHUNK, 8), jnp.float32)

  deg8 = _sc_degree(dst2, z8, ones8)

  t1 = _tc_call(
      _enc_body,
      [_row_spec(12), _row_spec(8), _full_spec(12, D), _full_spec(1, D),
       _full_spec(D, D)],
      D,
      (x, deg8, W_enc, b_enc.reshape(1, D), W_g1))

  S1 = _sc_agg(src2, dst2, t1, z32)

  t2 = _tc_call(
      _mid_body,
      [_row_spec(D), _row_spec(D), _row_spec(8), _full_spec(1, D),
       _full_spec(1, D), _full_spec(1, D), _full_spec(D, D)],
      D,
      (S1, t1, deg8, b_g1.reshape(1, D), ln1_g.reshape(1, D),
       ln1_b.reshape(1, D), W_g2))

  S2 = _sc_agg(src2, dst2, t2, z32)

  W_cat = jnp.concatenate([W_q, W_f, W_v], axis=1)
  b_cat = jnp.concatenate([b_q, b_f, b_v]).reshape(1, -1)
  out = _tc_call(
      _out_body,
      [_row_spec(D), _row_spec(D), _row_spec(8), _full_spec(1, D),
       _full_spec(1, D), _full_spec(1, D), _full_spec(D, D), _full_spec(1, D),
       _full_spec(D, 22), _full_spec(1, 22)],
      22,
      (S2, t2, deg8, b_g2.reshape(1, D), ln2_g.reshape(1, D),
       ln2_b.reshape(1, D), W_ref, b_ref.reshape(1, D), W_cat, b_cat))
  return out


# speedup vs baseline: 9.8072x; 1.1056x over previous
import functools

import jax
import jax.numpy as jnp
from jax import lax
from jax.experimental import pallas as pl
from jax.experimental.pallas import tpu as pltpu
from jax.experimental.pallas import tpu_sc as plsc

N = 100000
E = 1600000
D = 32
EPS = 1e-5

NC = 2
NS = 16
HALF = N // NC
TBL = 50048
RPT = TBL // NS
WB_LAST = HALF - (NS - 1) * RPT
ZR = 136
NZ = RPT // ZR

CHUNK = 80
CPB = 8
BLK = CPB * CHUNK
EROWS = 20480
PAD_E = EROWS * CHUNK - E
ROWS_PT = EROWS // NS
NBLK = ROWS_PT // CPB

_MESH = plsc.VectorSubcoreMesh(
    core_axis_name="c", subcore_axis_name="s", num_cores=NC, num_subcores=NS)
_SC_PARAMS = pltpu.CompilerParams(use_tc_tiling_on_sc=False)


def _dst_local(ebuf, dloc, k, nbase):
  r = ebuf.at[k]
  w = dloc.at[k]
  for j in range(CHUNK // 16):
    v = r[pl.ds(CHUNK + j * 16, 16)]
    lo = v - nbase
    ok = (lo >= 0) & (lo < HALF)
    w[pl.ds(j * 16, 16)] = jnp.where(ok, lo, HALF)


def _zero_table(agg, zv, z_h, s):
  pltpu.sync_copy(z_h, zv)
  for t in range(NZ):
    pltpu.sync_copy(zv, agg.at[pl.ds(s * RPT + t * ZR, ZR)])


def _writeback(agg, out_h, s, nbase):
  @pl.when(s < NS - 1)
  def _():
    pltpu.sync_copy(agg.at[pl.ds(s * RPT, RPT)],
                    out_h.at[pl.ds(nbase + s * RPT, RPT)])

  @pl.when(s == NS - 1)
  def _():
    pltpu.sync_copy(agg.at[pl.ds((NS - 1) * RPT, WB_LAST)],
                    out_h.at[pl.ds(nbase + (NS - 1) * RPT, WB_LAST)])


@functools.partial(
    pl.kernel,
    out_type=jax.ShapeDtypeStruct((N, 8), jnp.float32),
    mesh=_MESH,
    compiler_params=_SC_PARAMS,
    scratch_types=[
        pltpu.VMEM_SHARED((TBL, 8), jnp.float32),
        pltpu.VMEM((ZR, 8), jnp.float32),
        pltpu.VMEM((CHUNK, 8), jnp.float32),
        pltpu.VMEM((CPB, 2 * CHUNK), jnp.int32),
        pltpu.VMEM((CPB, CHUNK), jnp.int32),
        pltpu.SemaphoreType.DMA,
    ])
def _sc_degree(e2_h, z_h, ones_h, deg_h, agg, zv, ov, ebuf, dloc, sem):
  c = lax.axis_index("c")
  s = lax.axis_index("s")
  nbase = c * HALF
  pltpu.sync_copy(ones_h, ov)
  _zero_table(agg, zv, z_h, s)
  plsc.subcore_barrier()

  def blk(b, carry):
    roff = s * ROWS_PT + b * CPB
    pltpu.sync_copy(e2_h.at[pl.ds(roff, CPB)], ebuf)
    for k in range(CPB):
      _dst_local(ebuf, dloc, k, nbase)
    descs = [pltpu.async_copy(ov, agg.at[dloc.at[k]], sem, add=True)
             for k in range(CPB)]
    for d in descs:
      d.wait()
    return carry

  lax.fori_loop(0, NBLK, blk, 0)
  plsc.subcore_barrier()
  _writeback(agg, deg_h, s, nbase)


@functools.partial(
    pl.kernel,
    out_type=jax.ShapeDtypeStruct((N, D), jnp.float32),
    mesh=_MESH,
    compiler_params=_SC_PARAMS,
    scratch_types=[
        pltpu.VMEM_SHARED((TBL, D), jnp.float32),
        pltpu.VMEM((ZR, D), jnp.float32),
        pltpu.VMEM((CPB, 2 * CHUNK), jnp.int32),
        pltpu.VMEM((CPB, CHUNK), jnp.int32),
        pltpu.VMEM((CHUNK, D), jnp.float32),
        pltpu.VMEM((CHUNK, D), jnp.float32),
        pltpu.SemaphoreType.DMA,
        pltpu.SemaphoreType.DMA,
        pltpu.SemaphoreType.DMA,
        pltpu.SemaphoreType.DMA,
    ])
def _sc_agg(e2_h, t_h, z_h, S_h, agg, zv, ebuf, dloc, rows0, rows1, gs0, gs1,
            ss0, ss1):
  c = lax.axis_index("c")
  s = lax.axis_index("s")
  nbase = c * HALF
  rows = [rows0, rows1]
  gsem = [gs0, gs1]
  ssem = [ss0, ss1]
  _zero_table(agg, zv, z_h, s)
  plsc.subcore_barrier()

  def blk(b, carry):
    roff = s * ROWS_PT + b * CPB
    pltpu.sync_copy(e2_h.at[pl.ds(roff, CPB)], ebuf)
    for k in range(CPB):
      _dst_local(ebuf, dloc, k, nbase)

    gd = [None, None]
    sd = [None, None]

    def fire(k):
      gd[k % 2] = pltpu.async_copy(
          t_h.at[ebuf.at[k, pl.ds(0, CHUNK)]], rows[k % 2], gsem[k % 2])

    fire(0)
    for k in range(CPB):
      if k + 1 < CPB:
        if k >= 1:
          sd[(k + 1) % 2].wait()
        fire(k + 1)
      gd[k % 2].wait()
      sd[k % 2] = pltpu.async_copy(
          rows[k % 2], agg.at[dloc.at[k]], ssem[k % 2], add=True)
    sd[(CPB - 2) % 2].wait()
    sd[(CPB - 1) % 2].wait()
    return carry

  lax.fori_loop(0, NBLK, blk, 0)
  plsc.subcore_barrier()
  _writeback(agg, S_h, s, nbase)


R = 10000


def _dis(deg_blk):
  return lax.rsqrt(deg_blk[:, 0:1] + 1.0)


def _enc_body(x_r, deg_r, we_r, be_r, wg_r, t1_r):
  h = jnp.dot(x_r[...], we_r[...], preferred_element_type=jnp.float32)
  h = h + be_r[...]
  t1_r[...] = jnp.dot(h, wg_r[...],
                      preferred_element_type=jnp.float32) * _dis(deg_r[...])


def _layer_norm(u, g, b):
  mu = jnp.mean(u, axis=-1, keepdims=True)
  var = jnp.mean((u - mu) ** 2, axis=-1, keepdims=True)
  return (u - mu) * lax.rsqrt(var + EPS) * g + b


def _mid_body(S_r, t_r, deg_r, bg_r, g_r, b_r, wg2_r, t2_r):
  dis = _dis(deg_r[...])
  u = (S_r[...] + t_r[...]) * dis + bg_r[...]
  h = jnp.maximum(_layer_norm(u, g_r[...], b_r[...]), 0.0)
  t2_r[...] = jnp.dot(h, wg2_r[...], preferred_element_type=jnp.float32) * dis


def _out_body(S_r, t_r, deg_r, bg_r, g_r, b_r, wref_r, bref_r, wcat_r, bcat_r,
              o_r):
  dis = _dis(deg_r[...])
  u = (S_r[...] + t_r[...]) * dis + bg_r[...]
  h = jnp.maximum(_layer_norm(u, g_r[...], b_r[...]), 0.0)
  belief = jnp.maximum(
      jnp.dot(h, wref_r[...], preferred_element_type=jnp.float32) + bref_r[...],
      0.0)
  o_r[...] = jnp.dot(belief, wcat_r[...],
                     preferred_element_type=jnp.float32) + bcat_r[...]


def _row_spec(cols):
  return pl.BlockSpec((R, cols), lambda i: (i, 0))


def _full_spec(r, c):
  return pl.BlockSpec((r, c), lambda i: (0, 0))


def _tc_call(body, in_specs, out_cols, args):
  return pl.pallas_call(
      body,
      grid=(N // R,),
      in_specs=in_specs,
      out_specs=_row_spec(out_cols),
      out_shape=jax.ShapeDtypeStruct((N, out_cols), jnp.float32),
  )(*args)


def kernel(x, edge_index, W_enc, b_enc, W_g1, b_g1, ln1_g, ln1_b, W_g2, b_g2,
           ln2_g, ln2_b, W_ref, b_ref, W_q, b_q, W_f, b_f, W_v, b_v):
  src = edge_index[0].astype(jnp.int32)
  dst = edge_index[1].astype(jnp.int32)
  src2 = jnp.concatenate([src, jnp.zeros((PAD_E,), jnp.int32)]
                         ).reshape(EROWS, CHUNK)
  dst2 = jnp.concatenate([dst, jnp.full((PAD_E,), N, jnp.int32)]
                         ).reshape(EROWS, CHUNK)
  edges2 = jnp.concatenate([src2, dst2], axis=1)
  z32 = jnp.zeros((ZR, D), jnp.float32)
  z8 = jnp.zeros((ZR, 8), jnp.float32)
  ones8 = jnp.ones((CHUNK, 8), jnp.float32)

  deg8 = _sc_degree(edges2, z8, ones8)

  t1 = _tc_call(
      _enc_body,
      [_row_spec(12), _row_spec(8), _full_spec(12, D), _full_spec(1, D),
       _full_spec(D, D)],
      D,
      (x, deg8, W_enc, b_enc.reshape(1, D), W_g1))

  S1 = _sc_agg(edges2, t1, z32)

  t2 = _tc_call(
      _mid_body,
      [_row_spec(D), _row_spec(D), _row_spec(8), _full_spec(1, D),
       _full_spec(1, D), _full_spec(1, D), _full_spec(D, D)],
      D,
      (S1, t1, deg8, b_g1.reshape(1, D), ln1_g.reshape(1, D),
       ln1_b.reshape(1, D), W_g2))

  S2 = _sc_agg(edges2, t2, z32)

  W_cat = jnp.concatenate([W_q, W_f, W_v], axis=1)
  b_cat = jnp.concatenate([b_q, b_f, b_v]).reshape(1, -1)
  out = _tc_call(
      _out_body,
      [_row_spec(D), _row_spec(D), _row_spec(8), _full_spec(1, D),
       _full_spec(1, D), _full_spec(1, D), _full_spec(D, D), _full_spec(1, D),
       _full_spec(D, 22), _full_spec(1, 22)],
      22,
      (S2, t2, deg8, b_g2.reshape(1, D), ln2_g.reshape(1, D),
       ln2_b.reshape(1, D), W_ref, b_ref.reshape(1, D), W_cat, b_cat))
  return out


# speedup vs baseline: 9.9933x; 1.0190x over previous
import functools

import jax
import jax.numpy as jnp
from jax import lax
from jax.experimental import pallas as pl
from jax.experimental.pallas import tpu as pltpu
from jax.experimental.pallas import tpu_sc as plsc

N = 100000
E = 1600000
D = 32
EPS = 1e-5

NC = 2
NS = 16
HALF = N // NC
TBL = 50048
RPT = TBL // NS
WB_LAST = HALF - (NS - 1) * RPT
ZR = 136
NZ = RPT // ZR

CHUNK = 128
CPB = 8
NBUF = 4
EROWS = 12800
PAD_E = EROWS * CHUNK - E
ROWS_PT = EROWS // NS
NBLK = ROWS_PT // CPB

_MESH = plsc.VectorSubcoreMesh(
    core_axis_name="c", subcore_axis_name="s", num_cores=NC, num_subcores=NS)
_SC_PARAMS = pltpu.CompilerParams(use_tc_tiling_on_sc=False)


def _dst_local(ebuf, dloc, k, nbase):
  r = ebuf.at[k]
  w = dloc.at[k]
  for j in range(CHUNK // 16):
    v = r[pl.ds(CHUNK + j * 16, 16)]
    lo = v - nbase
    ok = (lo >= 0) & (lo < HALF)
    w[pl.ds(j * 16, 16)] = jnp.where(ok, lo, HALF)


def _zero_table(agg, zv, z_h, s):
  pltpu.sync_copy(z_h, zv)
  for t in range(NZ):
    pltpu.sync_copy(zv, agg.at[pl.ds(s * RPT + t * ZR, ZR)])


def _writeback(agg, out_h, s, nbase):
  @pl.when(s < NS - 1)
  def _():
    pltpu.sync_copy(agg.at[pl.ds(s * RPT, RPT)],
                    out_h.at[pl.ds(nbase + s * RPT, RPT)])

  @pl.when(s == NS - 1)
  def _():
    pltpu.sync_copy(agg.at[pl.ds((NS - 1) * RPT, WB_LAST)],
                    out_h.at[pl.ds(nbase + (NS - 1) * RPT, WB_LAST)])


@functools.partial(
    pl.kernel,
    out_type=jax.ShapeDtypeStruct((N, 8), jnp.float32),
    mesh=_MESH,
    compiler_params=_SC_PARAMS,
    scratch_types=[
        pltpu.VMEM_SHARED((TBL, 8), jnp.float32),
        pltpu.VMEM((ZR, 8), jnp.float32),
        pltpu.VMEM((CHUNK, 8), jnp.float32),
        pltpu.VMEM((CPB, 2 * CHUNK), jnp.int32),
        pltpu.VMEM((CPB, CHUNK), jnp.int32),
        pltpu.SemaphoreType.DMA,
    ])
def _sc_degree(e2_h, z_h, ones_h, deg_h, agg, zv, ov, ebuf, dloc, sem):
  c = lax.axis_index("c")
  s = lax.axis_index("s")
  nbase = c * HALF
  pltpu.sync_copy(ones_h, ov)
  _zero_table(agg, zv, z_h, s)
  plsc.subcore_barrier()

  def blk(b, carry):
    roff = s * ROWS_PT + b * CPB
    pltpu.sync_copy(e2_h.at[pl.ds(roff, CPB)], ebuf)
    for k in range(CPB):
      _dst_local(ebuf, dloc, k, nbase)
    descs = [pltpu.async_copy(ov, agg.at[dloc.at[k]], sem, add=True)
             for k in range(CPB)]
    for d in descs:
      d.wait()
    return carry

  lax.fori_loop(0, NBLK, blk, 0)
  plsc.subcore_barrier()
  _writeback(agg, deg_h, s, nbase)


@functools.partial(
    pl.kernel,
    out_type=jax.ShapeDtypeStruct((N, D), jnp.float32),
    mesh=_MESH,
    compiler_params=_SC_PARAMS,
    scratch_types=[
        pltpu.VMEM_SHARED((TBL, D), jnp.float32),
        pltpu.VMEM((ZR, D), jnp.float32),
        pltpu.VMEM((CPB, 2 * CHUNK), jnp.int32),
        pltpu.VMEM((CPB, CHUNK), jnp.int32),
    ] + [pltpu.VMEM((CHUNK, D), jnp.float32) for _ in range(NBUF)]
      + [pltpu.SemaphoreType.DMA for _ in range(2 * NBUF)])
def _sc_agg(e2_h, t_h, z_h, S_h, agg, zv, ebuf, dloc, *bufs):
  rows = list(bufs[:NBUF])
  gsem = list(bufs[NBUF:2 * NBUF])
  ssem = list(bufs[2 * NBUF:])
  c = lax.axis_index("c")
  s = lax.axis_index("s")
  nbase = c * HALF
  _zero_table(agg, zv, z_h, s)
  plsc.subcore_barrier()

  def blk(b, carry):
    roff = s * ROWS_PT + b * CPB
    pltpu.sync_copy(e2_h.at[pl.ds(roff, CPB)], ebuf)
    for k in range(CPB):
      _dst_local(ebuf, dloc, k, nbase)

    gd = [None] * NBUF
    sd = [None] * NBUF

    def fire(k):
      i = k % NBUF
      gd[i] = pltpu.async_copy(
          t_h.at[ebuf.at[k, pl.ds(0, CHUNK)]], rows[i], gsem[i])

    for k in range(min(NBUF - 1, CPB)):
      fire(k)
    for k in range(CPB):
      nk = k + NBUF - 1
      if nk < CPB:
        if sd[nk % NBUF] is not None:
          sd[nk % NBUF].wait()
        fire(nk)
      gd[k % NBUF].wait()
      sd[k % NBUF] = pltpu.async_copy(
          rows[k % NBUF], agg.at[dloc.at[k]], ssem[k % NBUF], add=True)
    for i in range(NBUF):
      k = CPB - NBUF + i
      if k >= 0 and sd[k % NBUF] is not None:
        sd[k % NBUF].wait()
        sd[k % NBUF] = None
    return carry

  lax.fori_loop(0, NBLK, blk, 0)
  plsc.subcore_barrier()
  _writeback(agg, S_h, s, nbase)


R = 10000


def _dis(deg_blk):
  return lax.rsqrt(deg_blk[:, 0:1] + 1.0)


def _enc_body(x_r, deg_r, we_r, be_r, wg_r, t1_r):
  h = jnp.dot(x_r[...], we_r[...], preferred_element_type=jnp.float32)
  h = h + be_r[...]
  t1_r[...] = jnp.dot(h, wg_r[...],
                      preferred_element_type=jnp.float32) * _dis(deg_r[...])


def _layer_norm(u, g, b):
  mu = jnp.mean(u, axis=-1, keepdims=True)
  var = jnp.mean((u - mu) ** 2, axis=-1, keepdims=True)
  return (u - mu) * lax.rsqrt(var + EPS) * g + b


def _mid_body(S_r, t_r, deg_r, bg_r, g_r, b_r, wg2_r, t2_r):
  dis = _dis(deg_r[...])
  u = (S_r[...] + t_r[...]) * dis + bg_r[...]
  h = jnp.maximum(_layer_norm(u, g_r[...], b_r[...]), 0.0)
  t2_r[...] = jnp.dot(h, wg2_r[...], preferred_element_type=jnp.float32) * dis


def _out_body(S_r, t_r, deg_r, bg_r, g_r, b_r, wref_r, bref_r, wcat_r, bcat_r,
              o_r):
  dis = _dis(deg_r[...])
  u = (S_r[...] + t_r[...]) * dis + bg_r[...]
  h = jnp.maximum(_layer_norm(u, g_r[...], b_r[...]), 0.0)
  belief = jnp.maximum(
      jnp.dot(h, wref_r[...], preferred_element_type=jnp.float32) + bref_r[...],
      0.0)
  o_r[...] = jnp.dot(belief, wcat_r[...],
                     preferred_element_type=jnp.float32) + bcat_r[...]


def _row_spec(cols):
  return pl.BlockSpec((R, cols), lambda i: (i, 0))


def _full_spec(r, c):
  return pl.BlockSpec((r, c), lambda i: (0, 0))


def _tc_call(body, in_specs, out_cols, args):
  return pl.pallas_call(
      body,
      grid=(N // R,),
      in_specs=in_specs,
      out_specs=_row_spec(out_cols),
      out_shape=jax.ShapeDtypeStruct((N, out_cols), jnp.float32),
  )(*args)


def kernel(x, edge_index, W_enc, b_enc, W_g1, b_g1, ln1_g, ln1_b, W_g2, b_g2,
           ln2_g, ln2_b, W_ref, b_ref, W_q, b_q, W_f, b_f, W_v, b_v):
  src = edge_index[0].astype(jnp.int32)
  dst = edge_index[1].astype(jnp.int32)
  src2 = jnp.concatenate([src, jnp.zeros((PAD_E,), jnp.int32)]
                         ).reshape(EROWS, CHUNK)
  dst2 = jnp.concatenate([dst, jnp.full((PAD_E,), N, jnp.int32)]
                         ).reshape(EROWS, CHUNK)
  edges2 = jnp.concatenate([src2, dst2], axis=1)
  z32 = jnp.zeros((ZR, D), jnp.float32)
  z8 = jnp.zeros((ZR, 8), jnp.float32)
  ones8 = jnp.ones((CHUNK, 8), jnp.float32)

  deg8 = _sc_degree(edges2, z8, ones8)

  t1 = _tc_call(
      _enc_body,
      [_row_spec(12), _row_spec(8), _full_spec(12, D), _full_spec(1, D),
       _full_spec(D, D)],
      D,
      (x, deg8, W_enc, b_enc.reshape(1, D), W_g1))

  S1 = _sc_agg(edges2, t1, z32)

  t2 = _tc_call(
      _mid_body,
      [_row_spec(D), _row_spec(D), _row_spec(8), _full_spec(1, D),
       _full_spec(1, D), _full_spec(1, D), _full_spec(D, D)],
      D,
      (S1, t1, deg8, b_g1.reshape(1, D), ln1_g.reshape(1, D),
       ln1_b.reshape(1, D), W_g2))

  S2 = _sc_agg(edges2, t2, z32)

  W_cat = jnp.concatenate([W_q, W_f, W_v], axis=1)
  b_cat = jnp.concatenate([b_q, b_f, b_v]).reshape(1, -1)
  out = _tc_call(
      _out_body,
      [_row_spec(D), _row_spec(D), _row_spec(8), _full_spec(1, D),
       _full_spec(1, D), _full_spec(1, D), _full_spec(D, D), _full_spec(1, D),
       _full_spec(D, 22), _full_spec(1, 22)],
      22,
      (S2, t2, deg8, b_g2.reshape(1, D), ln2_g.reshape(1, D),
       ln2_b.reshape(1, D), W_ref, b_ref.reshape(1, D), W_cat, b_cat))
  return out


# speedup vs baseline: 20.9350x; 2.0949x over previous
import functools

import jax
import jax.numpy as jnp
from jax import lax
from jax.experimental import pallas as pl
from jax.experimental.pallas import tpu as pltpu
from jax.experimental.pallas import tpu_sc as plsc

N = 100000
E = 1600000
D = 32
HD = D // 2
EPS = 1e-5

NC = 2
NS = 16
NW = NC * NS

TBL = 100096
RPT = TBL // NS
WB_LAST = N - (NS - 1) * RPT
ZR = 782
NZ = RPT // ZR

CHUNK = 128
CPB = 8
NBUF = 4
EROWS = 12800
PAD_E = EROWS * CHUNK - E
ROWS_PT = EROWS // NS
NBLK = ROWS_PT // CPB
ROWS_PW = EROWS // NW
DBLK = ROWS_PW // CPB

_MESH = plsc.VectorSubcoreMesh(
    core_axis_name="c", subcore_axis_name="s", num_cores=NC, num_subcores=NS)
_SC_PARAMS = pltpu.CompilerParams(use_tc_tiling_on_sc=False)


def _zero_table(agg, zv, z_h, s):
  pltpu.sync_copy(z_h, zv)
  for t in range(NZ):
    pltpu.sync_copy(zv, agg.at[pl.ds(s * RPT + t * ZR, ZR)])


def _writeback(agg, out_h, s):
  @pl.when(s < NS - 1)
  def _():
    pltpu.sync_copy(agg.at[pl.ds(s * RPT, RPT)],
                    out_h.at[pl.ds(s * RPT, RPT)])

  @pl.when(s == NS - 1)
  def _():
    pltpu.sync_copy(agg.at[pl.ds((NS - 1) * RPT, WB_LAST)],
                    out_h.at[pl.ds((NS - 1) * RPT, WB_LAST)])


@functools.partial(
    pl.kernel,
    out_type=(jax.ShapeDtypeStruct((N, 1), jnp.float32),
              jax.ShapeDtypeStruct((N, 1), jnp.float32)),
    mesh=_MESH,
    compiler_params=_SC_PARAMS,
    scratch_types=[
        pltpu.VMEM_SHARED((TBL, 1), jnp.float32),
        pltpu.VMEM((RPT, 1), jnp.float32),
        pltpu.VMEM((CHUNK, 1), jnp.float32),
        pltpu.VMEM((CPB, CHUNK), jnp.int32),
        pltpu.SemaphoreType.DMA,
    ])
def _sc_degree(dst2_h, z_h, ones_h, d0_h, d1_h, agg, zv, ov, draw, sem):
  c = lax.axis_index("c")
  s = lax.axis_index("s")
  pltpu.sync_copy(ones_h, ov)
  pltpu.sync_copy(z_h, zv)
  pltpu.sync_copy(zv, agg.at[pl.ds(s * RPT, RPT)])
  plsc.subcore_barrier()

  wid = c * NS + s

  def blk(b, carry):
    roff = wid * ROWS_PW + b * CPB
    pltpu.sync_copy(dst2_h.at[pl.ds(roff, CPB)], draw)
    descs = [pltpu.async_copy(ov, agg.at[draw.at[k]], sem, add=True)
             for k in range(CPB)]
    for d in descs:
      d.wait()
    return carry

  lax.fori_loop(0, DBLK, blk, 0)
  plsc.subcore_barrier()

  @pl.when(c == 0)
  def _():
    _writeback(agg, d0_h, s)

  @pl.when(c == 1)
  def _():
    _writeback(agg, d1_h, s)


@functools.partial(
    pl.kernel,
    out_type=(jax.ShapeDtypeStruct((N, HD), jnp.float32),
              jax.ShapeDtypeStruct((N, HD), jnp.float32)),
    mesh=_MESH,
    compiler_params=_SC_PARAMS,
    scratch_types=[
        pltpu.VMEM_SHARED((TBL, HD), jnp.float32),
        pltpu.VMEM((ZR, HD), jnp.float32),
        pltpu.VMEM((CPB, CHUNK), jnp.int32),
        pltpu.VMEM((CPB, CHUNK), jnp.int32),
    ] + [pltpu.VMEM((CHUNK, HD), jnp.float32) for _ in range(NBUF)]
      + [pltpu.SemaphoreType.DMA for _ in range(2 * NBUF)])
def _sc_agg(src2_h, dst2_h, tlo_h, thi_h, z_h, Slo_h, Shi_h, agg, zv, sidx,
            draw, *bufs):
  rows = list(bufs[:NBUF])
  gsem = list(bufs[NBUF:2 * NBUF])
  ssem = list(bufs[2 * NBUF:])
  c = lax.axis_index("c")
  s = lax.axis_index("s")

  def run(t_h, S_h):
    _zero_table(agg, zv, z_h, s)
    plsc.subcore_barrier()

    def blk(b, carry):
      roff = s * ROWS_PT + b * CPB
      pltpu.sync_copy(src2_h.at[pl.ds(roff, CPB)], sidx)
      pltpu.sync_copy(dst2_h.at[pl.ds(roff, CPB)], draw)

      gd = [None] * NBUF
      sd = [None] * NBUF

      def fire(k):
        i = k % NBUF
        gd[i] = pltpu.async_copy(t_h.at[sidx.at[k]], rows[i], gsem[i])

      for k in range(min(NBUF - 1, CPB)):
        fire(k)
      for k in range(CPB):
        nk = k + NBUF - 1
        if nk < CPB:
          if sd[nk % NBUF] is not None:
            sd[nk % NBUF].wait()
          fire(nk)
        gd[k % NBUF].wait()
        sd[k % NBUF] = pltpu.async_copy(
            rows[k % NBUF], agg.at[draw.at[k]], ssem[k % NBUF], add=True)
      for i in range(NBUF):
        k = CPB - NBUF + i
        if k >= 0 and sd[k % NBUF] is not None:
          sd[k % NBUF].wait()
      return carry

    lax.fori_loop(0, NBLK, blk, 0)
    plsc.subcore_barrier()
    _writeback(agg, S_h, s)

  @pl.when(c == 0)
  def _():
    run(tlo_h, Slo_h)

  @pl.when(c == 1)
  def _():
    run(thi_h, Shi_h)


R = 2000


def _dis(d0, d1):
  return lax.rsqrt(d0 + d1 + 1.0)


def _enc_body(x_r, d0_r, d1_r, we_r, be_r, wg_r, tlo_r, thi_r):
  h = jnp.dot(x_r[...], we_r[...], preferred_element_type=jnp.float32)
  h = h + be_r[...]
  t = jnp.dot(h, wg_r[...],
              preferred_element_type=jnp.float32) * _dis(d0_r[...], d1_r[...])
  tlo_r[...] = t[:, :HD]
  thi_r[...] = t[:, HD:]


def _layer_norm(u, g, b):
  mu = jnp.mean(u, axis=-1, keepdims=True)
  var = jnp.mean((u - mu) ** 2, axis=-1, keepdims=True)
  return (u - mu) * lax.rsqrt(var + EPS) * g + b


def _mid_body(Sl_r, Sh_r, tl_r, th_r, d0_r, d1_r, bg_r, g_r, b_r, wg2_r,
              t2lo_r, t2hi_r):
  dis = _dis(d0_r[...], d1_r[...])
  S = jnp.concatenate([Sl_r[...], Sh_r[...]], axis=-1)
  t = jnp.concatenate([tl_r[...], th_r[...]], axis=-1)
  u = (S + t) * dis + bg_r[...]
  h = jnp.maximum(_layer_norm(u, g_r[...], b_r[...]), 0.0)
  t2 = jnp.dot(h, wg2_r[...], preferred_element_type=jnp.float32) * dis
  t2lo_r[...] = t2[:, :HD]
  t2hi_r[...] = t2[:, HD:]


def _out_body(Sl_r, Sh_r, tl_r, th_r, d0_r, d1_r, bg_r, g_r, b_r, wref_r,
              bref_r, wcat_r, bcat_r, o_r):
  dis = _dis(d0_r[...], d1_r[...])
  S = jnp.concatenate([Sl_r[...], Sh_r[...]], axis=-1)
  t = jnp.concatenate([tl_r[...], th_r[...]], axis=-1)
  u = (S + t) * dis + bg_r[...]
  h = jnp.maximum(_layer_norm(u, g_r[...], b_r[...]), 0.0)
  belief = jnp.maximum(
      jnp.dot(h, wref_r[...], preferred_element_type=jnp.float32) + bref_r[...],
      0.0)
  o_r[...] = jnp.dot(belief, wcat_r[...],
                     preferred_element_type=jnp.float32) + bcat_r[...]


def _row_spec(cols):
  return pl.BlockSpec((R, cols), lambda i: (i, 0))


def _full_spec(r, c):
  return pl.BlockSpec((r, c), lambda i: (0, 0))


def _tc_call(body, in_specs, out_cols, args):
  if isinstance(out_cols, tuple):
    out_specs = [_row_spec(cc) for cc in out_cols]
    out_shape = [jax.ShapeDtypeStruct((N, cc), jnp.float32)
                 for cc in out_cols]
  else:
    out_specs = _row_spec(out_cols)
    out_shape = jax.ShapeDtypeStruct((N, out_cols), jnp.float32)
  return pl.pallas_call(
      body,
      grid=(N // R,),
      in_specs=in_specs,
      out_specs=out_specs,
      out_shape=out_shape,
  )(*args)


def kernel(x, edge_index, W_enc, b_enc, W_g1, b_g1, ln1_g, ln1_b, W_g2, b_g2,
           ln2_g, ln2_b, W_ref, b_ref, W_q, b_q, W_f, b_f, W_v, b_v):
  src = edge_index[0].astype(jnp.int32)
  dst = edge_index[1].astype(jnp.int32)
  src2 = jnp.concatenate([src, jnp.zeros((PAD_E,), jnp.int32)]
                         ).reshape(EROWS, CHUNK)
  dst2 = jnp.concatenate([dst, jnp.full((PAD_E,), N, jnp.int32)]
                         ).reshape(EROWS, CHUNK)
  z16 = jnp.zeros((ZR, HD), jnp.float32)
  z1 = jnp.zeros((RPT, 1), jnp.float32)
  ones1 = jnp.ones((CHUNK, 1), jnp.float32)

  d0, d1 = _sc_degree(dst2, z1, ones1)

  t1lo, t1hi = _tc_call(
      _enc_body,
      [_row_spec(12), _row_spec(1), _row_spec(1), _full_spec(12, D),
       _full_spec(1, D), _full_spec(D, D)],
      (HD, HD),
      (x, d0, d1, W_enc, b_enc.reshape(1, D), W_g1))

  S1lo, S1hi = _sc_agg(src2, dst2, t1lo, t1hi, z16)

  t2lo, t2hi = _tc_call(
      _mid_body,
      [_row_spec(HD), _row_spec(HD), _row_spec(HD), _row_spec(HD),
       _row_spec(1), _row_spec(1), _full_spec(1, D), _full_spec(1, D),
       _full_spec(1, D), _full_spec(D, D)],
      (HD, HD),
      (S1lo, S1hi, t1lo, t1hi, d0, d1, b_g1.reshape(1, D),
       ln1_g.reshape(1, D), ln1_b.reshape(1, D), W_g2))

  S2lo, S2hi = _sc_agg(src2, dst2, t2lo, t2hi, z16)

  W_cat = jnp.concatenate([W_q, W_f, W_v], axis=1)
  b_cat = jnp.concatenate([b_q, b_f, b_v]).reshape(1, -1)
  out = _tc_call(
      _out_body,
      [_row_spec(HD), _row_spec(HD), _row_spec(HD), _row_spec(HD),
       _row_spec(1), _row_spec(1), _full_spec(1, D), _full_spec(1, D),
       _full_spec(1, D), _full_spec(D, D), _full_spec(1, D),
       _full_spec(D, 22), _full_spec(1, 22)],
      22,
      (S2lo, S2hi, t2lo, t2hi, d0, d1, b_g2.reshape(1, D),
       ln2_g.reshape(1, D), ln2_b.reshape(1, D), W_ref, b_ref.reshape(1, D),
       W_cat, b_cat))
  return out


# speedup vs baseline: 21.1115x; 1.0084x over previous
import functools

import jax
import jax.numpy as jnp
from jax import lax
from jax.experimental import pallas as pl
from jax.experimental.pallas import tpu as pltpu
from jax.experimental.pallas import tpu_sc as plsc

N = 100000
E = 1600000
D = 32
HD = D // 2
EPS = 1e-5

NC = 2
NS = 16
NW = NC * NS

TBL = 100096
RPT = TBL // NS
WB_LAST = N - (NS - 1) * RPT
ZR = 782
NZ = RPT // ZR

CHUNK = 128
CPB = 8
NBUF = 4
EROWS = 12800
PAD_E = EROWS * CHUNK - E
ROWS_PT = EROWS // NS
NBLK = ROWS_PT // CPB
ROWS_PW = EROWS // NW
DBLK = ROWS_PW // CPB

_MESH = plsc.VectorSubcoreMesh(
    core_axis_name="c", subcore_axis_name="s", num_cores=NC, num_subcores=NS)
_SC_PARAMS = pltpu.CompilerParams(use_tc_tiling_on_sc=False)


def _zero_table(agg, zv, z_h, s):
  pltpu.sync_copy(z_h, zv)
  for t in range(NZ):
    pltpu.sync_copy(zv, agg.at[pl.ds(s * RPT + t * ZR, ZR)])


def _writeback(agg, out_h, s):
  @pl.when(s < NS - 1)
  def _():
    pltpu.sync_copy(agg.at[pl.ds(s * RPT, RPT)],
                    out_h.at[pl.ds(s * RPT, RPT)])

  @pl.when(s == NS - 1)
  def _():
    pltpu.sync_copy(agg.at[pl.ds((NS - 1) * RPT, WB_LAST)],
                    out_h.at[pl.ds((NS - 1) * RPT, WB_LAST)])


@functools.partial(
    pl.kernel,
    out_type=(jax.ShapeDtypeStruct((N, 8), jnp.float32),
              jax.ShapeDtypeStruct((N, 8), jnp.float32)),
    mesh=_MESH,
    compiler_params=_SC_PARAMS,
    scratch_types=[
        pltpu.VMEM_SHARED((TBL, 8), jnp.float32),
        pltpu.VMEM((RPT, 8), jnp.float32),
        pltpu.VMEM((CHUNK, 8), jnp.float32),
        pltpu.VMEM((CPB, CHUNK), jnp.int32),
        pltpu.SemaphoreType.DMA,
    ])
def _sc_degree(dst2_h, z_h, ones_h, d0_h, d1_h, agg, zv, ov, draw, sem):
  c = lax.axis_index("c")
  s = lax.axis_index("s")
  pltpu.sync_copy(ones_h, ov)
  pltpu.sync_copy(z_h, zv)
  pltpu.sync_copy(zv, agg.at[pl.ds(s * RPT, RPT)])
  plsc.subcore_barrier()

  wid = c * NS + s

  def blk(b, carry):
    roff = wid * ROWS_PW + b * CPB
    pltpu.sync_copy(dst2_h.at[pl.ds(roff, CPB)], draw)
    descs = [pltpu.async_copy(ov, agg.at[draw.at[k]], sem, add=True)
             for k in range(CPB)]
    for d in descs:
      d.wait()
    return carry

  lax.fori_loop(0, DBLK, blk, 0)
  plsc.subcore_barrier()

  @pl.when(c == 0)
  def _():
    _writeback(agg, d0_h, s)

  @pl.when(c == 1)
  def _():
    _writeback(agg, d1_h, s)


@functools.partial(
    pl.kernel,
    out_type=(jax.ShapeDtypeStruct((N, HD), jnp.float32),
              jax.ShapeDtypeStruct((N, HD), jnp.float32)),
    mesh=_MESH,
    compiler_params=_SC_PARAMS,
    scratch_types=[
        pltpu.VMEM_SHARED((TBL, HD), jnp.float32),
        pltpu.VMEM((ZR, HD), jnp.float32),
        pltpu.VMEM((CPB, CHUNK), jnp.int32),
        pltpu.VMEM((CPB, CHUNK), jnp.int32),
    ] + [pltpu.VMEM((CHUNK, HD), jnp.float32) for _ in range(NBUF)]
      + [pltpu.SemaphoreType.DMA for _ in range(2 * NBUF)])
def _sc_agg(src2_h, dst2_h, tlo_h, thi_h, z_h, Slo_h, Shi_h, agg, zv, sidx,
            draw, *bufs):
  rows = list(bufs[:NBUF])
  gsem = list(bufs[NBUF:2 * NBUF])
  ssem = list(bufs[2 * NBUF:])
  c = lax.axis_index("c")
  s = lax.axis_index("s")

  def run(t_h, S_h):
    _zero_table(agg, zv, z_h, s)
    plsc.subcore_barrier()

    def blk(b, carry):
      roff = s * ROWS_PT + b * CPB
      pltpu.sync_copy(src2_h.at[pl.ds(roff, CPB)], sidx)
      pltpu.sync_copy(dst2_h.at[pl.ds(roff, CPB)], draw)

      gd = [None] * NBUF
      sd = [None] * NBUF

      def fire(k):
        i = k % NBUF
        gd[i] = pltpu.async_copy(t_h.at[sidx.at[k]], rows[i], gsem[i])

      for k in range(min(NBUF - 1, CPB)):
        fire(k)
      for k in range(CPB):
        nk = k + NBUF - 1
        if nk < CPB:
          if sd[nk % NBUF] is not None:
            sd[nk % NBUF].wait()
          fire(nk)
        gd[k % NBUF].wait()
        sd[k % NBUF] = pltpu.async_copy(
            rows[k % NBUF], agg.at[draw.at[k]], ssem[k % NBUF], add=True)
      for i in range(NBUF):
        k = CPB - NBUF + i
        if k >= 0 and sd[k % NBUF] is not None:
          sd[k % NBUF].wait()
      return carry

    lax.fori_loop(0, NBLK, blk, 0)
    plsc.subcore_barrier()
    _writeback(agg, S_h, s)

  @pl.when(c == 0)
  def _():
    run(tlo_h, Slo_h)

  @pl.when(c == 1)
  def _():
    run(thi_h, Shi_h)


R = 2000


def _dis(d0, d1):
  return lax.rsqrt(d0[:, 0:1] + d1[:, 0:1] + 1.0)


def _enc_body(x_r, d0_r, d1_r, we_r, be_r, wg_r, tlo_r, thi_r):
  h = jnp.dot(x_r[...], we_r[...], preferred_element_type=jnp.float32)
  h = h + be_r[...]
  t = jnp.dot(h, wg_r[...],
              preferred_element_type=jnp.float32) * _dis(d0_r[...], d1_r[...])
  tlo_r[...] = t[:, :HD]
  thi_r[...] = t[:, HD:]


def _layer_norm(u, g, b):
  mu = jnp.mean(u, axis=-1, keepdims=True)
  var = jnp.mean((u - mu) ** 2, axis=-1, keepdims=True)
  return (u - mu) * lax.rsqrt(var + EPS) * g + b


def _mid_body(Sl_r, Sh_r, tl_r, th_r, d0_r, d1_r, bg_r, g_r, b_r, wg2_r,
              t2lo_r, t2hi_r):
  dis = _dis(d0_r[...], d1_r[...])
  S = jnp.concatenate([Sl_r[...], Sh_r[...]], axis=-1)
  t = jnp.concatenate([tl_r[...], th_r[...]], axis=-1)
  u = (S + t) * dis + bg_r[...]
  h = jnp.maximum(_layer_norm(u, g_r[...], b_r[...]), 0.0)
  t2 = jnp.dot(h, wg2_r[...], preferred_element_type=jnp.float32) * dis
  t2lo_r[...] = t2[:, :HD]
  t2hi_r[...] = t2[:, HD:]


def _out_body(Sl_r, Sh_r, tl_r, th_r, d0_r, d1_r, bg_r, g_r, b_r, wref_r,
              bref_r, wcat_r, bcat_r, o_r):
  dis = _dis(d0_r[...], d1_r[...])
  S = jnp.concatenate([Sl_r[...], Sh_r[...]], axis=-1)
  t = jnp.concatenate([tl_r[...], th_r[...]], axis=-1)
  u = (S + t) * dis + bg_r[...]
  h = jnp.maximum(_layer_norm(u, g_r[...], b_r[...]), 0.0)
  belief = jnp.maximum(
      jnp.dot(h, wref_r[...], preferred_element_type=jnp.float32) + bref_r[...],
      0.0)
  o_r[...] = jnp.dot(belief, wcat_r[...],
                     preferred_element_type=jnp.float32) + bcat_r[...]


def _row_spec(cols):
  return pl.BlockSpec((R, cols), lambda i: (i, 0))


def _full_spec(r, c):
  return pl.BlockSpec((r, c), lambda i: (0, 0))


def _tc_call(body, in_specs, out_cols, args):
  if isinstance(out_cols, tuple):
    out_specs = [_row_spec(cc) for cc in out_cols]
    out_shape = [jax.ShapeDtypeStruct((N, cc), jnp.float32)
                 for cc in out_cols]
  else:
    out_specs = _row_spec(out_cols)
    out_shape = jax.ShapeDtypeStruct((N, out_cols), jnp.float32)
  return pl.pallas_call(
      body,
      grid=(N // R,),
      in_specs=in_specs,
      out_specs=out_specs,
      out_shape=out_shape,
  )(*args)


def kernel(x, edge_index, W_enc, b_enc, W_g1, b_g1, ln1_g, ln1_b, W_g2, b_g2,
           ln2_g, ln2_b, W_ref, b_ref, W_q, b_q, W_f, b_f, W_v, b_v):
  src = edge_index[0].astype(jnp.int32)
  dst = edge_index[1].astype(jnp.int32)
  src2 = jnp.concatenate([src, jnp.zeros((PAD_E,), jnp.int32)]
                         ).reshape(EROWS, CHUNK)
  dst2 = jnp.concatenate([dst, jnp.full((PAD_E,), N, jnp.int32)]
                         ).reshape(EROWS, CHUNK)
  z16 = jnp.zeros((ZR, HD), jnp.float32)
  z1 = jnp.zeros((RPT, 8), jnp.float32)
  ones1 = jnp.ones((CHUNK, 8), jnp.float32)

  d0, d1 = _sc_degree(dst2, z1, ones1)

  t1lo, t1hi = _tc_call(
      _enc_body,
      [_row_spec(12), _row_spec(8), _row_spec(8), _full_spec(12, D),
       _full_spec(1, D), _full_spec(D, D)],
      (HD, HD),
      (x, d0, d1, W_enc, b_enc.reshape(1, D), W_g1))

  S1lo, S1hi = _sc_agg(src2, dst2, t1lo, t1hi, z16)

  t2lo, t2hi = _tc_call(
      _mid_body,
      [_row_spec(HD), _row_spec(HD), _row_spec(HD), _row_spec(HD),
       _row_spec(8), _row_spec(8), _full_spec(1, D), _full_spec(1, D),
       _full_spec(1, D), _full_spec(D, D)],
      (HD, HD),
      (S1lo, S1hi, t1lo, t1hi, d0, d1, b_g1.reshape(1, D),
       ln1_g.reshape(1, D), ln1_b.reshape(1, D), W_g2))

  S2lo, S2hi = _sc_agg(src2, dst2, t2lo, t2hi, z16)

  W_cat = jnp.concatenate([W_q, W_f, W_v], axis=1)
  b_cat = jnp.concatenate([b_q, b_f, b_v]).reshape(1, -1)
  out = _tc_call(
      _out_body,
      [_row_spec(HD), _row_spec(HD), _row_spec(HD), _row_spec(HD),
       _row_spec(8), _row_spec(8), _full_spec(1, D), _full_spec(1, D),
       _full_spec(1, D), _full_spec(D, D), _full_spec(1, D),
       _full_spec(D, 22), _full_spec(1, 22)],
      22,
      (S2lo, S2hi, t2lo, t2hi, d0, d1, b_g2.reshape(1, D),
       ln2_g.reshape(1, D), ln2_b.reshape(1, D), W_ref, b_ref.reshape(1, D),
       W_cat, b_cat))
  return out
